# baseline (speedup 1.0000x reference)
import jax
import jax.numpy as jnp
from jax import lax
from jax.experimental import pallas as pl
from jax.experimental.pallas import tpu as pltpu
from jax.experimental.pallas import tpu_sc as plsc

N = 10000
E = 320000
D = 128
LAT = 64
NC, NS, L = 2, 16, 16
NW = NC * NS
EPW = E // NW
C = 80
NCH = EPW // C
NBUF = 5
HD = D // NC
EPS_ = E // NS
NCH2 = EPS_ // C
C2 = 80
NCH3 = 250
PAD = NCH3 * C2 - EPS_
NGRP = NCH3 // NBUF
NPAIR = NGRP // 2
NTRASH = 8
ROWS_PW = N // NS
NPAD = 10240
DSL = NPAD // NS

_mesh = plsc.VectorSubcoreMesh(
    core_axis_name="c", subcore_axis_name="s", num_cores=NC, num_subcores=NS)


def _deg_body(s_hbm, r_hbm, out_hbm, s_idx, r_idx, zbuf, ones_buf,
              deg_s_sh, deg_r_sh):
    c = lax.axis_index("c")
    s = lax.axis_index("s")
    z16 = jnp.zeros((L,), jnp.float32)
    o16 = jnp.full((L,), 1.0, jnp.float32)
    for k in range(DSL // L):
        zbuf[pl.ds(k * L, L)] = z16
    for k in range(C // L):
        ones_buf[pl.ds(k * L, L)] = o16
    pltpu.sync_copy(zbuf, deg_s_sh.at[pl.ds(s * DSL, DSL)])
    pltpu.sync_copy(zbuf, deg_r_sh.at[pl.ds(s * DSL, DSL)])
    pltpu.sync_copy(s_hbm.at[c, s], s_idx)
    pltpu.sync_copy(r_hbm.at[c, s], r_idx)
    plsc.subcore_barrier()

    def body(j, carry):
        pltpu.sync_copy(ones_buf, deg_s_sh.at[s_idx.at[j]], add=True)
        pltpu.sync_copy(ones_buf, deg_r_sh.at[r_idx.at[j]], add=True)
        return carry

    lax.fori_loop(0, NCH, body, 0)
    plsc.subcore_barrier()
    pltpu.sync_copy(deg_s_sh.at[pl.ds(s * DSL, DSL)], out_hbm.at[c, 0, s])
    pltpu.sync_copy(deg_r_sh.at[pl.ds(s * DSL, DSL)], out_hbm.at[c, 1, s])


_deg_kernel = pl.kernel(
    _deg_body,
    out_type=jax.ShapeDtypeStruct((NC, 2, NS, DSL), jnp.float32),
    mesh=_mesh,
    scratch_types=[
        pltpu.VMEM((NCH, C), jnp.int32),
        pltpu.VMEM((NCH, C), jnp.int32),
        pltpu.VMEM((DSL,), jnp.float32),
        pltpu.VMEM((C,), jnp.float32),
        pltpu.VMEM_SHARED((NPAD,), jnp.float32),
        pltpu.VMEM_SHARED((NPAD,), jnp.float32),
    ],
)


def _prop_body(tab_hbm, s_hbm, r_hbm, out_hbm,
               si0, ri0, si1, ri1,
               p00, p01, p02, p03, p04, p10, p11, p12, p13, p14,
               acc_sh, gsem0, gsem1, ssem0, ssem1, isem0, isem1):
    P = ((p00, p01, p02, p03, p04), (p10, p11, p12, p13, p14))
    SI = (si0, si1)
    RI = (ri0, ri1)
    gsem = (gsem0, gsem1)
    ssem = (ssem0, ssem1)
    isem = (isem0, isem1)
    c = lax.axis_index("c")
    s = lax.axis_index("s")
    z16 = jnp.zeros((L,), jnp.float32)

    def zs(i, carry):
        for k in range(HD // L):
            P[0][0][i, pl.ds(k * L, L)] = z16
        return carry

    lax.fori_loop(0, C2, zs, 0)
    base = s * ROWS_PW
    for k in range(ROWS_PW // C2):
        pltpu.sync_copy(P[0][0], acc_sh.at[pl.ds(base + k * C2, C2)])
    rem = ROWS_PW - (ROWS_PW // C2) * C2
    pltpu.sync_copy(P[0][0].at[pl.ds(0, rem)],
                    acc_sh.at[pl.ds(base + (ROWS_PW // C2) * C2, rem)])
    tab_c = tab_hbm.at[c]
    s_row = s_hbm.at[s]
    r_row = r_hbm.at[s]

    def fetch_idx(par, j0):
        pltpu.async_copy(s_row.at[pl.ds(j0, NBUF)], SI[par], isem[par])
        pltpu.async_copy(r_row.at[pl.ds(j0, NBUF)], RI[par], isem[par])

    def wait_idx(par):
        pltpu.make_async_copy(s_row.at[pl.ds(0, NBUF)], SI[par],
                              isem[par]).wait()
        pltpu.make_async_copy(r_row.at[pl.ds(0, NBUF)], RI[par],
                              isem[par]).wait()

    def fire_gathers(par):
        for b in range(NBUF):
            pltpu.async_copy(tab_c.at[SI[par].at[b]], P[par][b], gsem[par])

    def wait_gathers(par):
        for b in range(NBUF):
            pltpu.make_async_copy(tab_c.at[pl.ds(0, C2)], P[par][b],
                                  gsem[par]).wait()

    def do_scatters(par):
        for b in range(NBUF):
            pltpu.async_copy(
                P[par][b], acc_sh.at[RI[par].at[b]], ssem[par], add=True)

    def wait_scatters(par):
        for b in range(NBUF):
            pltpu.make_async_copy(tab_c.at[pl.ds(0, C2)], P[par][b],
                                  ssem[par]).wait()

    pltpu.sync_copy(s_row.at[pl.ds(0, NBUF)], SI[0])
    pltpu.sync_copy(r_row.at[pl.ds(0, NBUF)], RI[0])
    fetch_idx(1, NBUF)
    plsc.subcore_barrier()
    fire_gathers(0)
    JMAX = NCH3 - NBUF

    def pair(i, carry):
        a = 2 * i
        wait_gathers(0)
        wait_idx(1)
        fire_gathers(1)
        do_scatters(0)
        wait_scatters(0)
        fetch_idx(0, jnp.minimum((a + 2) * NBUF, JMAX))
        wait_gathers(1)
        wait_idx(0)
        fire_gathers(0)
        do_scatters(1)
        wait_scatters(1)
        fetch_idx(1, jnp.minimum((a + 3) * NBUF, JMAX))
        return carry

    lax.fori_loop(0, NPAIR, pair, 0)
    wait_gathers(0)
    wait_idx(1)
    plsc.subcore_barrier()
    pltpu.sync_copy(acc_sh.at[pl.ds(base, ROWS_PW)], out_hbm.at[c, s])


_prop_kernel = pl.kernel(
    _prop_body,
    out_type=jax.ShapeDtypeStruct((NC, NS, ROWS_PW, HD), jnp.float32),
    mesh=_mesh,
    scratch_types=(
        [pltpu.VMEM((NBUF, C2), jnp.int32) for _ in range(4)]
        + [pltpu.VMEM((C2, HD), jnp.float32) for _ in range(2 * NBUF)]
        + [pltpu.VMEM_SHARED((N + NTRASH, HD), jnp.float32)]
        + [pltpu.SemaphoreType.DMA for _ in range(6)]
    ),
    compiler_params=pltpu.CompilerParams(use_tc_tiling_on_sc=False),
)


def _prop(tab, s2, r2):
    return _prop_kernel(tab, s2, r2).reshape(NC, N, HD)


def _elu(u):
    return jnp.where(u > 0, u, jnp.exp(u) - 1.0)


def _cat(A_ref):
    return jnp.concatenate([A_ref[0], A_ref[1]], axis=1)


def _mm(h, W_ref, b_ref):
    return jnp.dot(h, W_ref[...], preferred_element_type=jnp.float32) + b_ref[...]


def _stage1a_body(x_ref, Wa_ref, Wb_ref, ba_ref, bb_ref, eu_ref):
    x = x_ref[...]
    eu_ref[0] = _elu(_mm(x, Wa_ref, ba_ref))
    eu_ref[1] = _elu(_mm(x, Wb_ref, bb_ref))


def _stage1b_body(eu_ref, dsa, dsb, dra, drb,
                  t1_ref, rss_ref, rrs_ref, rsl_ref, rrl_ref):
    deg_s = dsa[...] + dsb[...]
    deg_r = dra[...] + drb[...]
    rss = lax.rsqrt(deg_s + 1.0)
    rss_ref[...] = rss
    rrs_ref[...] = lax.rsqrt(deg_r + 1.0)
    rsl_ref[...] = lax.rsqrt(jnp.maximum(deg_s, 1.0))
    rrl_ref[...] = lax.rsqrt(jnp.maximum(deg_r, 1.0))
    t1_ref[0] = eu_ref[0] * rss
    t1_ref[1] = eu_ref[1] * rss


def _stage2_body(A_ref, t1_ref, rrs_ref, rsl_ref,
                 Wm_ref, bm_ref, Wl_ref, bl_ref, t23_ref):
    h = jnp.concatenate([A_ref[0] + t1_ref[0], A_ref[1] + t1_ref[1]],
                        axis=1) * rrs_ref[...]
    rsl = rsl_ref[...]
    t23_ref[0] = _mm(h, Wm_ref, bm_ref) * rsl
    t23_ref[1] = _mm(h, Wl_ref, bl_ref) * rsl


def _stage3_body(A_ref, rrl_ref, eps_ref, Wa_ref, Wb_ref, ba_ref, bb_ref,
                 rss_ref, mean_ref, ls_ref, t4_ref):
    rrl = rrl_ref[...]
    mean = A_ref[0] * rrl
    lsd = A_ref[1] * rrl
    mean_ref[...] = mean
    ls_ref[...] = lsd
    z = mean + jnp.exp(lsd) * eps_ref[...]
    rss = rss_ref[...]
    t4_ref[0] = _elu(_mm(z, Wa_ref, ba_ref)) * rss
    t4_ref[1] = _elu(_mm(z, Wb_ref, bb_ref)) * rss


def _stage4_body(A_ref, t4_ref, rrs_ref, Wa_ref, Wb_ref, ba_ref, bb_ref,
                 rsl_ref, t5_ref):
    d = jnp.concatenate([A_ref[0] + t4_ref[0], A_ref[1] + t4_ref[1]],
                        axis=1) * rrs_ref[...]
    rsl = rsl_ref[...]
    t5_ref[0] = _mm(d, Wa_ref, ba_ref) * rsl
    t5_ref[1] = _mm(d, Wb_ref, bb_ref) * rsl


def _stage5_body(A_ref, rrl_ref, out_ref):
    out_ref[...] = _cat(A_ref) * rrl_ref[...]


_f32 = jnp.float32
BN = 2000
_G = (N // BN,)
_vec = jax.ShapeDtypeStruct((N, 1), _f32)
_tab = jax.ShapeDtypeStruct((NC, N, HD), _f32)


def _bs_rows(w):
    return pl.BlockSpec((BN, w), lambda i: (i, 0))


def _bs_tab():
    return pl.BlockSpec((NC, BN, HD), lambda i: (0, i, 0))


def _bs_full(shape):
    return pl.BlockSpec(shape, lambda i: tuple(0 for _ in shape))


_vec_spec = _bs_rows(1)
_tab_out = jax.ShapeDtypeStruct((NC, N, HD), _f32)

_stage1a = pl.pallas_call(
    _stage1a_body,
    grid=_G,
    in_specs=[_bs_rows(D), _bs_full((D, HD)), _bs_full((D, HD)),
              _bs_full((1, HD)), _bs_full((1, HD))],
    out_specs=_bs_tab(),
    out_shape=_tab)
_stage1b = pl.pallas_call(
    _stage1b_body,
    grid=_G,
    in_specs=[_bs_tab(), _vec_spec, _vec_spec, _vec_spec, _vec_spec],
    out_specs=(_bs_tab(), _vec_spec, _vec_spec, _vec_spec, _vec_spec),
    out_shape=(_tab, _vec, _vec, _vec, _vec))
_stage2 = pl.pallas_call(
    _stage2_body,
    grid=_G,
    in_specs=[_bs_tab(), _bs_tab(), _vec_spec, _vec_spec,
              _bs_full((D, HD)), _bs_full((1, HD)),
              _bs_full((D, HD)), _bs_full((1, HD))],
    out_specs=_bs_tab(),
    out_shape=_tab)
_stage3 = pl.pallas_call(
    _stage3_body,
    grid=_G,
    in_specs=[_bs_tab(), _vec_spec, _bs_rows(LAT),
              _bs_full((LAT, HD)), _bs_full((LAT, HD)),
              _bs_full((1, HD)), _bs_full((1, HD)), _vec_spec],
    out_specs=(_bs_rows(LAT), _bs_rows(LAT), _bs_tab()),
    out_shape=(jax.ShapeDtypeStruct((N, LAT), _f32),
               jax.ShapeDtypeStruct((N, LAT), _f32), _tab),
)
_stage4 = pl.pallas_call(
    _stage4_body,
    grid=_G,
    in_specs=[_bs_tab(), _bs_tab(), _vec_spec,
              _bs_full((D, HD)), _bs_full((D, HD)),
              _bs_full((1, HD)), _bs_full((1, HD)), _vec_spec],
    out_specs=_bs_tab(),
    out_shape=_tab)
_stage5 = pl.pallas_call(
    _stage5_body,
    grid=_G,
    in_specs=[_bs_tab(), _vec_spec],
    out_specs=_bs_rows(D),
    out_shape=jax.ShapeDtypeStruct((N, D), _f32))


def kernel(x, edge_index, W_h, b_h, W_mean, b_mean, W_ls, b_ls,
           W_dh, b_dh, W_do, b_do, eps):
    s = edge_index[0].astype(jnp.int32)
    r = edge_index[1].astype(jnp.int32)
    s2 = s.reshape(NC, NS, NCH, C)
    r2 = r.reshape(NC, NS, NCH, C)
    pad = ((0, 0), (0, PAD))
    s3 = jnp.pad(s.reshape(NS, EPS_), pad).reshape(NS, NCH3, C2)
    r3 = jnp.pad(r.reshape(NS, EPS_), pad,
                 constant_values=N).reshape(NS, NCH3, C2)
    degp = _deg_kernel(s2, r2).reshape(NC, 2, NPAD)
    dsa = degp[0, 0, :N, None]
    dsb = degp[1, 0, :N, None]
    dra = degp[0, 1, :N, None]
    drb = degp[1, 1, :N, None]
    bh = b_h[None, :]
    eu = _stage1a(x, W_h[:, :HD], W_h[:, HD:], bh[:, :HD], bh[:, HD:])
    t1, rss, rrs, rsl, rrl = _stage1b(eu, dsa, dsb, dra, drb)
    A1 = _prop(t1, s3, r3)
    t23 = _stage2(A1, t1, rrs, rsl, W_mean, b_mean[None, :],
                  W_ls, b_ls[None, :])
    A23 = _prop(t23, s3, r3)
    bdh = b_dh[None, :]
    mean, log_std, t4 = _stage3(
        A23, rrl, eps, W_dh[:, :HD], W_dh[:, HD:], bdh[:, :HD], bdh[:, HD:],
        rss)
    A4 = _prop(t4, s3, r3)
    bdo = b_do[None, :]
    t5 = _stage4(A4, t4, rrs, W_do[:, :HD], W_do[:, HD:],
                 bdo[:, :HD], bdo[:, HD:], rsl)
    A5 = _prop(t5, s3, r3)
    out = _stage5(A5, rrl)
    return (mean, log_std, out)


# speedup vs baseline: 1.0078x; 1.0078x over previous
import jax
import jax.numpy as jnp
from jax import lax
from jax.experimental import pallas as pl
from jax.experimental.pallas import tpu as pltpu
from jax.experimental.pallas import tpu_sc as plsc

N = 10000
E = 320000
D = 128
LAT = 64
NC, NS, L = 2, 16, 16
NW = NC * NS
EPW = E // NW
C = 80
NCH = EPW // C
NBUF = 5
HD = D // NC
EPS_ = E // NS
NCH2 = EPS_ // C
C2 = 80
NCH3 = 250
PAD = NCH3 * C2 - EPS_
NGRP = NCH3 // NBUF
NPAIR = NGRP // 2
NTRASH = 8
ROWS_PW = N // NS
NPAD = 10240
DSL = NPAD // NS

_mesh = plsc.VectorSubcoreMesh(
    core_axis_name="c", subcore_axis_name="s", num_cores=NC, num_subcores=NS)


def _deg_body(s_hbm, r_hbm, out_hbm, s_idx, r_idx, zbuf, ones_buf,
              deg_s_sh, deg_r_sh):
    c = lax.axis_index("c")
    s = lax.axis_index("s")
    z16 = jnp.zeros((L,), jnp.float32)
    o16 = jnp.full((L,), 1.0, jnp.float32)
    for k in range(DSL // L):
        zbuf[pl.ds(k * L, L)] = z16
    for k in range(C // L):
        ones_buf[pl.ds(k * L, L)] = o16
    pltpu.sync_copy(zbuf, deg_s_sh.at[pl.ds(s * DSL, DSL)])
    pltpu.sync_copy(zbuf, deg_r_sh.at[pl.ds(s * DSL, DSL)])
    pltpu.sync_copy(s_hbm.at[c, s], s_idx)
    pltpu.sync_copy(r_hbm.at[c, s], r_idx)
    plsc.subcore_barrier()

    def body(j, carry):
        pltpu.sync_copy(ones_buf, deg_s_sh.at[s_idx.at[j]], add=True)
        pltpu.sync_copy(ones_buf, deg_r_sh.at[r_idx.at[j]], add=True)
        return carry

    lax.fori_loop(0, NCH, body, 0)
    plsc.subcore_barrier()
    pltpu.sync_copy(deg_s_sh.at[pl.ds(s * DSL, DSL)], out_hbm.at[c, 0, s])
    pltpu.sync_copy(deg_r_sh.at[pl.ds(s * DSL, DSL)], out_hbm.at[c, 1, s])


_deg_kernel = pl.kernel(
    _deg_body,
    out_type=jax.ShapeDtypeStruct((NC, 2, NS, DSL), jnp.float32),
    mesh=_mesh,
    scratch_types=[
        pltpu.VMEM((NCH, C), jnp.int32),
        pltpu.VMEM((NCH, C), jnp.int32),
        pltpu.VMEM((DSL,), jnp.float32),
        pltpu.VMEM((C,), jnp.float32),
        pltpu.VMEM_SHARED((NPAD,), jnp.float32),
        pltpu.VMEM_SHARED((NPAD,), jnp.float32),
    ],
    compiler_params=pltpu.CompilerParams(use_tc_tiling_on_sc=False),
)


def _prop_body(tab_hbm, s_hbm, r_hbm, out_hbm,
               si0, ri0, si1, ri1,
               p00, p01, p02, p03, p04, p10, p11, p12, p13, p14,
               acc_sh, gsem0, gsem1, ssem0, ssem1, isem0, isem1):
    P = ((p00, p01, p02, p03, p04), (p10, p11, p12, p13, p14))
    SI = (si0, si1)
    RI = (ri0, ri1)
    gsem = (gsem0, gsem1)
    ssem = (ssem0, ssem1)
    isem = (isem0, isem1)
    c = lax.axis_index("c")
    s = lax.axis_index("s")
    z16 = jnp.zeros((L,), jnp.float32)

    def zs(i, carry):
        for k in range(HD // L):
            P[0][0][i, pl.ds(k * L, L)] = z16
        return carry

    lax.fori_loop(0, C2, zs, 0)
    base = s * ROWS_PW
    for k in range(ROWS_PW // C2):
        pltpu.sync_copy(P[0][0], acc_sh.at[pl.ds(base + k * C2, C2)])
    rem = ROWS_PW - (ROWS_PW // C2) * C2
    pltpu.sync_copy(P[0][0].at[pl.ds(0, rem)],
                    acc_sh.at[pl.ds(base + (ROWS_PW // C2) * C2, rem)])
    tab_c = tab_hbm.at[c]
    s_row = s_hbm.at[s]
    r_row = r_hbm.at[s]

    def fetch_idx(par, j0):
        pltpu.async_copy(s_row.at[pl.ds(j0, NBUF)], SI[par], isem[par])
        pltpu.async_copy(r_row.at[pl.ds(j0, NBUF)], RI[par], isem[par])

    def wait_idx(par):
        pltpu.make_async_copy(s_row.at[pl.ds(0, NBUF)], SI[par],
                              isem[par]).wait()
        pltpu.make_async_copy(r_row.at[pl.ds(0, NBUF)], RI[par],
                              isem[par]).wait()

    def fire_gathers(par):
        for b in range(NBUF):
            pltpu.async_copy(tab_c.at[SI[par].at[b]], P[par][b], gsem[par])

    def wait_gathers(par):
        for b in range(NBUF):
            pltpu.make_async_copy(tab_c.at[pl.ds(0, C2)], P[par][b],
                                  gsem[par]).wait()

    def do_scatters(par):
        for b in range(NBUF):
            pltpu.async_copy(
                P[par][b], acc_sh.at[RI[par].at[b]], ssem[par], add=True)

    def wait_scatters(par):
        for b in range(NBUF):
            pltpu.make_async_copy(tab_c.at[pl.ds(0, C2)], P[par][b],
                                  ssem[par]).wait()

    pltpu.sync_copy(s_row.at[pl.ds(0, NBUF)], SI[0])
    pltpu.sync_copy(r_row.at[pl.ds(0, NBUF)], RI[0])
    fetch_idx(1, NBUF)
    plsc.subcore_barrier()
    fire_gathers(0)
    JMAX = NCH3 - NBUF

    def pair(i, carry):
        a = 2 * i
        wait_gathers(0)
        wait_idx(1)
        fire_gathers(1)
        do_scatters(0)
        wait_scatters(0)
        fetch_idx(0, jnp.minimum((a + 2) * NBUF, JMAX))
        wait_gathers(1)
        wait_idx(0)
        fire_gathers(0)
        do_scatters(1)
        wait_scatters(1)
        fetch_idx(1, jnp.minimum((a + 3) * NBUF, JMAX))
        return carry

    lax.fori_loop(0, NPAIR, pair, 0)
    wait_gathers(0)
    wait_idx(1)
    plsc.subcore_barrier()
    pltpu.sync_copy(acc_sh.at[pl.ds(base, ROWS_PW)], out_hbm.at[c, s])


_prop_kernel = pl.kernel(
    _prop_body,
    out_type=jax.ShapeDtypeStruct((NC, NS, ROWS_PW, HD), jnp.float32),
    mesh=_mesh,
    scratch_types=(
        [pltpu.VMEM((NBUF, C2), jnp.int32) for _ in range(4)]
        + [pltpu.VMEM((C2, HD), jnp.float32) for _ in range(2 * NBUF)]
        + [pltpu.VMEM_SHARED((N + NTRASH, HD), jnp.float32)]
        + [pltpu.SemaphoreType.DMA for _ in range(6)]
    ),
    compiler_params=pltpu.CompilerParams(use_tc_tiling_on_sc=False),
)


def _prop(tab, s2, r2):
    return _prop_kernel(tab, s2, r2).reshape(NC, N, HD)


def _elu(u):
    return jnp.where(u > 0, u, jnp.exp(u) - 1.0)


def _cat(A_ref):
    return jnp.concatenate([A_ref[0], A_ref[1]], axis=1)


def _mm(h, W_ref, b_ref):
    return jnp.dot(h, W_ref[...], preferred_element_type=jnp.float32) + b_ref[...]


def _stage1_body(x_ref, Wa_ref, Wb_ref, ba_ref, bb_ref, dsa, dsb, dra, drb,
                 t1_ref, rss_ref, rrs_ref, rsl_ref, rrl_ref):
    deg_s = dsa[...] + dsb[...]
    deg_r = dra[...] + drb[...]
    rss = lax.rsqrt(deg_s + 1.0)
    rss_ref[...] = rss
    rrs_ref[...] = lax.rsqrt(deg_r + 1.0)
    rsl_ref[...] = lax.rsqrt(jnp.maximum(deg_s, 1.0))
    rrl_ref[...] = lax.rsqrt(jnp.maximum(deg_r, 1.0))
    x = x_ref[...]
    t1_ref[0] = _elu(_mm(x, Wa_ref, ba_ref)) * rss
    t1_ref[1] = _elu(_mm(x, Wb_ref, bb_ref)) * rss


def _stage2_body(A_ref, t1_ref, rrs_ref, rsl_ref,
                 Wm_ref, bm_ref, Wl_ref, bl_ref, t23_ref):
    h = jnp.concatenate([A_ref[0] + t1_ref[0], A_ref[1] + t1_ref[1]],
                        axis=1) * rrs_ref[...]
    rsl = rsl_ref[...]
    t23_ref[0] = _mm(h, Wm_ref, bm_ref) * rsl
    t23_ref[1] = _mm(h, Wl_ref, bl_ref) * rsl


def _stage3_body(A_ref, rrl_ref, eps_ref, Wa_ref, Wb_ref, ba_ref, bb_ref,
                 rss_ref, mean_ref, ls_ref, t4_ref):
    rrl = rrl_ref[...]
    mean = A_ref[0] * rrl
    lsd = A_ref[1] * rrl
    mean_ref[...] = mean
    ls_ref[...] = lsd
    z = mean + jnp.exp(lsd) * eps_ref[...]
    rss = rss_ref[...]
    t4_ref[0] = _elu(_mm(z, Wa_ref, ba_ref)) * rss
    t4_ref[1] = _elu(_mm(z, Wb_ref, bb_ref)) * rss


def _stage4_body(A_ref, t4_ref, rrs_ref, Wa_ref, Wb_ref, ba_ref, bb_ref,
                 rsl_ref, t5_ref):
    d = jnp.concatenate([A_ref[0] + t4_ref[0], A_ref[1] + t4_ref[1]],
                        axis=1) * rrs_ref[...]
    rsl = rsl_ref[...]
    t5_ref[0] = _mm(d, Wa_ref, ba_ref) * rsl
    t5_ref[1] = _mm(d, Wb_ref, bb_ref) * rsl


def _stage5_body(A_ref, rrl_ref, out_ref):
    out_ref[...] = _cat(A_ref) * rrl_ref[...]


_f32 = jnp.float32
BN = 2000
_G = (N // BN,)
_vec = jax.ShapeDtypeStruct((N, 1), _f32)
_tab = jax.ShapeDtypeStruct((NC, N, HD), _f32)


def _bs_rows(w):
    return pl.BlockSpec((BN, w), lambda i: (i, 0))


def _bs_tab():
    return pl.BlockSpec((NC, BN, HD), lambda i: (0, i, 0))


def _bs_full(shape):
    return pl.BlockSpec(shape, lambda i: tuple(0 for _ in shape))


_vec_spec = _bs_rows(1)
_tab_out = jax.ShapeDtypeStruct((NC, N, HD), _f32)

_stage1 = pl.pallas_call(
    _stage1_body,
    grid=_G,
    in_specs=[_bs_rows(D), _bs_full((D, HD)), _bs_full((D, HD)),
              _bs_full((1, HD)), _bs_full((1, HD)),
              _vec_spec, _vec_spec, _vec_spec, _vec_spec],
    out_specs=(_bs_tab(), _vec_spec, _vec_spec, _vec_spec, _vec_spec),
    out_shape=(_tab, _vec, _vec, _vec, _vec))
_stage2 = pl.pallas_call(
    _stage2_body,
    grid=_G,
    in_specs=[_bs_tab(), _bs_tab(), _vec_spec, _vec_spec,
              _bs_full((D, HD)), _bs_full((1, HD)),
              _bs_full((D, HD)), _bs_full((1, HD))],
    out_specs=_bs_tab(),
    out_shape=_tab)
_stage3 = pl.pallas_call(
    _stage3_body,
    grid=_G,
    in_specs=[_bs_tab(), _vec_spec, _bs_rows(LAT),
              _bs_full((LAT, HD)), _bs_full((LAT, HD)),
              _bs_full((1, HD)), _bs_full((1, HD)), _vec_spec],
    out_specs=(_bs_rows(LAT), _bs_rows(LAT), _bs_tab()),
    out_shape=(jax.ShapeDtypeStruct((N, LAT), _f32),
               jax.ShapeDtypeStruct((N, LAT), _f32), _tab),
)
_stage4 = pl.pallas_call(
    _stage4_body,
    grid=_G,
    in_specs=[_bs_tab(), _bs_tab(), _vec_spec,
              _bs_full((D, HD)), _bs_full((D, HD)),
              _bs_full((1, HD)), _bs_full((1, HD)), _vec_spec],
    out_specs=_bs_tab(),
    out_shape=_tab)
_stage5 = pl.pallas_call(
    _stage5_body,
    grid=_G,
    in_specs=[_bs_tab(), _vec_spec],
    out_specs=_bs_rows(D),
    out_shape=jax.ShapeDtypeStruct((N, D), _f32))


def kernel(x, edge_index, W_h, b_h, W_mean, b_mean, W_ls, b_ls,
           W_dh, b_dh, W_do, b_do, eps):
    s = edge_index[0].astype(jnp.int32)
    r = edge_index[1].astype(jnp.int32)
    s2 = s.reshape(NC, NS, NCH, C)
    r2 = r.reshape(NC, NS, NCH, C)
    pad = ((0, 0), (0, PAD))
    s3 = jnp.pad(s.reshape(NS, EPS_), pad).reshape(NS, NCH3, C2)
    r3 = jnp.pad(r.reshape(NS, EPS_), pad,
                 constant_values=N).reshape(NS, NCH3, C2)
    degp = _deg_kernel(s2, r2).reshape(NC, 2, NPAD)
    dsa = degp[0, 0, :N, None]
    dsb = degp[1, 0, :N, None]
    dra = degp[0, 1, :N, None]
    drb = degp[1, 1, :N, None]
    bh = b_h[None, :]
    t1, rss, rrs, rsl, rrl = _stage1(
        x, W_h[:, :HD], W_h[:, HD:], bh[:, :HD], bh[:, HD:],
        dsa, dsb, dra, drb)
    A1 = _prop(t1, s3, r3)
    t23 = _stage2(A1, t1, rrs, rsl, W_mean, b_mean[None, :],
                  W_ls, b_ls[None, :])
    A23 = _prop(t23, s3, r3)
    bdh = b_dh[None, :]
    mean, log_std, t4 = _stage3(
        A23, rrl, eps, W_dh[:, :HD], W_dh[:, HD:], bdh[:, :HD], bdh[:, HD:],
        rss)
    A4 = _prop(t4, s3, r3)
    bdo = b_do[None, :]
    t5 = _stage4(A4, t4, rrs, W_do[:, :HD], W_do[:, HD:],
                 bdo[:, :HD], bdo[:, HD:], rsl)
    A5 = _prop(t5, s3, r3)
    out = _stage5(A5, rrl)
    return (mean, log_std, out)


# speedup vs baseline: 1.0293x; 1.0213x over previous
import jax
import jax.numpy as jnp
from jax import lax
from jax.experimental import pallas as pl
from jax.experimental.pallas import tpu as pltpu
from jax.experimental.pallas import tpu_sc as plsc

N = 10000
E = 320000
D = 128
LAT = 64
NC, NS, L = 2, 16, 16
NW = NC * NS
EPW = E // NW
C = 80
NCH = EPW // C
NBUF = 5
HD = D // NC
EPS_ = E // NS
NCH2 = EPS_ // C
C2 = 80
NCH3 = 250
PAD = NCH3 * C2 - EPS_
NGRP = NCH3 // NBUF
NPAIR = NGRP // 2
NTRASH = 8
ROWS_PW = N // NS
NPAD = 10240
DSL = NPAD // NS

_mesh = plsc.VectorSubcoreMesh(
    core_axis_name="c", subcore_axis_name="s", num_cores=NC, num_subcores=NS)


def _deg_body(s_hbm, r_hbm, out_hbm, s_idx, r_idx, zbuf, ones_buf,
              deg_s_sh, deg_r_sh, dsem):
    c = lax.axis_index("c")
    s = lax.axis_index("s")
    z16 = jnp.zeros((L,), jnp.float32)
    o16 = jnp.full((L,), 1.0, jnp.float32)
    for k in range(DSL // L):
        zbuf[pl.ds(k * L, L)] = z16
    for k in range(C // L):
        ones_buf[pl.ds(k * L, L)] = o16
    pltpu.sync_copy(zbuf, deg_s_sh.at[pl.ds(s * DSL, DSL)])
    pltpu.sync_copy(zbuf, deg_r_sh.at[pl.ds(s * DSL, DSL)])
    pltpu.sync_copy(s_hbm.at[c, s], s_idx)
    pltpu.sync_copy(r_hbm.at[c, s], r_idx)
    plsc.subcore_barrier()

    def body(g, carry):
        cps = []
        for b in range(5):
            j = g * 5 + b
            cps.append(pltpu.async_copy(
                ones_buf, deg_s_sh.at[s_idx.at[j]], dsem, add=True))
            cps.append(pltpu.async_copy(
                ones_buf, deg_r_sh.at[r_idx.at[j]], dsem, add=True))
        for cp in cps:
            cp.wait()
        return carry

    lax.fori_loop(0, NCH // 5, body, 0)
    plsc.subcore_barrier()
    pltpu.sync_copy(deg_s_sh.at[pl.ds(s * DSL, DSL)], out_hbm.at[c, 0, s])
    pltpu.sync_copy(deg_r_sh.at[pl.ds(s * DSL, DSL)], out_hbm.at[c, 1, s])


_deg_kernel = pl.kernel(
    _deg_body,
    out_type=jax.ShapeDtypeStruct((NC, 2, NS, DSL), jnp.float32),
    mesh=_mesh,
    scratch_types=[
        pltpu.VMEM((NCH, C), jnp.int32),
        pltpu.VMEM((NCH, C), jnp.int32),
        pltpu.VMEM((DSL,), jnp.float32),
        pltpu.VMEM((C,), jnp.float32),
        pltpu.VMEM_SHARED((NPAD,), jnp.float32),
        pltpu.VMEM_SHARED((NPAD,), jnp.float32),
        pltpu.SemaphoreType.DMA,
    ],
    compiler_params=pltpu.CompilerParams(use_tc_tiling_on_sc=False),
)


def _prop_body(tab_hbm, s_hbm, r_hbm, out_hbm,
               si0, ri0, si1, ri1,
               p00, p01, p02, p03, p04, p10, p11, p12, p13, p14,
               acc_sh, gsem0, gsem1, ssem0, ssem1, isem0, isem1):
    P = ((p00, p01, p02, p03, p04), (p10, p11, p12, p13, p14))
    SI = (si0, si1)
    RI = (ri0, ri1)
    gsem = (gsem0, gsem1)
    ssem = (ssem0, ssem1)
    isem = (isem0, isem1)
    c = lax.axis_index("c")
    s = lax.axis_index("s")
    z16 = jnp.zeros((L,), jnp.float32)

    def zs(i, carry):
        for k in range(HD // L):
            P[0][0][i, pl.ds(k * L, L)] = z16
        return carry

    lax.fori_loop(0, C2, zs, 0)
    base = s * ROWS_PW
    for k in range(ROWS_PW // C2):
        pltpu.sync_copy(P[0][0], acc_sh.at[pl.ds(base + k * C2, C2)])
    rem = ROWS_PW - (ROWS_PW // C2) * C2
    pltpu.sync_copy(P[0][0].at[pl.ds(0, rem)],
                    acc_sh.at[pl.ds(base + (ROWS_PW // C2) * C2, rem)])
    tab_c = tab_hbm.at[c]
    s_row = s_hbm.at[s]
    r_row = r_hbm.at[s]

    def fetch_idx(par, j0):
        pltpu.async_copy(s_row.at[pl.ds(j0, NBUF)], SI[par], isem[par])
        pltpu.async_copy(r_row.at[pl.ds(j0, NBUF)], RI[par], isem[par])

    def wait_idx(par):
        pltpu.make_async_copy(s_row.at[pl.ds(0, NBUF)], SI[par],
                              isem[par]).wait()
        pltpu.make_async_copy(r_row.at[pl.ds(0, NBUF)], RI[par],
                              isem[par]).wait()

    def fire_gathers(par):
        for b in range(NBUF):
            pltpu.async_copy(tab_c.at[SI[par].at[b]], P[par][b], gsem[par])

    def wait_gathers(par):
        for b in range(NBUF):
            pltpu.make_async_copy(tab_c.at[pl.ds(0, C2)], P[par][b],
                                  gsem[par]).wait()

    def do_scatters(par):
        for b in range(NBUF):
            pltpu.async_copy(
                P[par][b], acc_sh.at[RI[par].at[b]], ssem[par], add=True)

    def wait_scatters(par):
        for b in range(NBUF):
            pltpu.make_async_copy(tab_c.at[pl.ds(0, C2)], P[par][b],
                                  ssem[par]).wait()

    pltpu.sync_copy(s_row.at[pl.ds(0, NBUF)], SI[0])
    pltpu.sync_copy(r_row.at[pl.ds(0, NBUF)], RI[0])
    fetch_idx(1, NBUF)
    plsc.subcore_barrier()
    fire_gathers(0)
    JMAX = NCH3 - NBUF

    def pair(i, carry):
        a = 2 * i
        wait_gathers(0)
        wait_idx(1)
        fire_gathers(1)
        do_scatters(0)
        wait_scatters(0)
        fetch_idx(0, jnp.minimum((a + 2) * NBUF, JMAX))
        wait_gathers(1)
        wait_idx(0)
        fire_gathers(0)
        do_scatters(1)
        wait_scatters(1)
        fetch_idx(1, jnp.minimum((a + 3) * NBUF, JMAX))
        return carry

    lax.fori_loop(0, NPAIR, pair, 0)
    wait_gathers(0)
    wait_idx(1)
    plsc.subcore_barrier()
    pltpu.sync_copy(acc_sh.at[pl.ds(base, ROWS_PW)], out_hbm.at[c, s])


_prop_kernel = pl.kernel(
    _prop_body,
    out_type=jax.ShapeDtypeStruct((NC, NS, ROWS_PW, HD), jnp.float32),
    mesh=_mesh,
    scratch_types=(
        [pltpu.VMEM((NBUF, C2), jnp.int32) for _ in range(4)]
        + [pltpu.VMEM((C2, HD), jnp.float32) for _ in range(2 * NBUF)]
        + [pltpu.VMEM_SHARED((N + NTRASH, HD), jnp.float32)]
        + [pltpu.SemaphoreType.DMA for _ in range(6)]
    ),
    compiler_params=pltpu.CompilerParams(use_tc_tiling_on_sc=False),
)


def _prop(tab, s2, r2):
    return _prop_kernel(tab, s2, r2).reshape(NC, N, HD)


def _elu(u):
    return jnp.where(u > 0, u, jnp.exp(u) - 1.0)


def _cat(A_ref):
    return jnp.concatenate([A_ref[0], A_ref[1]], axis=1)


def _mm(h, W_ref, b_ref):
    return jnp.dot(h, W_ref[...], preferred_element_type=jnp.float32) + b_ref[...]


def _stage1_body(x_ref, Wa_ref, Wb_ref, ba_ref, bb_ref, dsa, dsb, dra, drb,
                 t1_ref, rss_ref, rrs_ref, rsl_ref, rrl_ref):
    deg_s = dsa[...] + dsb[...]
    deg_r = dra[...] + drb[...]
    rss = lax.rsqrt(deg_s + 1.0)
    rss_ref[...] = rss
    rrs_ref[...] = lax.rsqrt(deg_r + 1.0)
    rsl_ref[...] = lax.rsqrt(jnp.maximum(deg_s, 1.0))
    rrl_ref[...] = lax.rsqrt(jnp.maximum(deg_r, 1.0))
    x = x_ref[...]
    t1_ref[0] = _elu(_mm(x, Wa_ref, ba_ref)) * rss
    t1_ref[1] = _elu(_mm(x, Wb_ref, bb_ref)) * rss


def _stage2_body(A_ref, t1_ref, rrs_ref, rsl_ref,
                 Wm_ref, bm_ref, Wl_ref, bl_ref, t23_ref):
    h = jnp.concatenate([A_ref[0] + t1_ref[0], A_ref[1] + t1_ref[1]],
                        axis=1) * rrs_ref[...]
    rsl = rsl_ref[...]
    t23_ref[0] = _mm(h, Wm_ref, bm_ref) * rsl
    t23_ref[1] = _mm(h, Wl_ref, bl_ref) * rsl


def _stage3_body(A_ref, rrl_ref, eps_ref, Wa_ref, Wb_ref, ba_ref, bb_ref,
                 rss_ref, mean_ref, ls_ref, t4_ref):
    rrl = rrl_ref[...]
    mean = A_ref[0] * rrl
    lsd = A_ref[1] * rrl
    mean_ref[...] = mean
    ls_ref[...] = lsd
    z = mean + jnp.exp(lsd) * eps_ref[...]
    rss = rss_ref[...]
    t4_ref[0] = _elu(_mm(z, Wa_ref, ba_ref)) * rss
    t4_ref[1] = _elu(_mm(z, Wb_ref, bb_ref)) * rss


def _stage4_body(A_ref, t4_ref, rrs_ref, Wa_ref, Wb_ref, ba_ref, bb_ref,
                 rsl_ref, t5_ref):
    d = jnp.concatenate([A_ref[0] + t4_ref[0], A_ref[1] + t4_ref[1]],
                        axis=1) * rrs_ref[...]
    rsl = rsl_ref[...]
    t5_ref[0] = _mm(d, Wa_ref, ba_ref) * rsl
    t5_ref[1] = _mm(d, Wb_ref, bb_ref) * rsl


def _stage5_body(A_ref, rrl_ref, out_ref):
    out_ref[...] = _cat(A_ref) * rrl_ref[...]


_f32 = jnp.float32
BN = 2000
_G = (N // BN,)
_vec = jax.ShapeDtypeStruct((N, 1), _f32)
_tab = jax.ShapeDtypeStruct((NC, N, HD), _f32)


def _bs_rows(w):
    return pl.BlockSpec((BN, w), lambda i: (i, 0))


def _bs_tab():
    return pl.BlockSpec((NC, BN, HD), lambda i: (0, i, 0))


def _bs_full(shape):
    return pl.BlockSpec(shape, lambda i: tuple(0 for _ in shape))


_vec_spec = _bs_rows(1)
_tab_out = jax.ShapeDtypeStruct((NC, N, HD), _f32)

_stage1 = pl.pallas_call(
    _stage1_body,
    grid=_G,
    in_specs=[_bs_rows(D), _bs_full((D, HD)), _bs_full((D, HD)),
              _bs_full((1, HD)), _bs_full((1, HD)),
              _vec_spec, _vec_spec, _vec_spec, _vec_spec],
    out_specs=(_bs_tab(), _vec_spec, _vec_spec, _vec_spec, _vec_spec),
    out_shape=(_tab, _vec, _vec, _vec, _vec))
_stage2 = pl.pallas_call(
    _stage2_body,
    grid=_G,
    in_specs=[_bs_tab(), _bs_tab(), _vec_spec, _vec_spec,
              _bs_full((D, HD)), _bs_full((1, HD)),
              _bs_full((D, HD)), _bs_full((1, HD))],
    out_specs=_bs_tab(),
    out_shape=_tab)
_stage3 = pl.pallas_call(
    _stage3_body,
    grid=_G,
    in_specs=[_bs_tab(), _vec_spec, _bs_rows(LAT),
              _bs_full((LAT, HD)), _bs_full((LAT, HD)),
              _bs_full((1, HD)), _bs_full((1, HD)), _vec_spec],
    out_specs=(_bs_rows(LAT), _bs_rows(LAT), _bs_tab()),
    out_shape=(jax.ShapeDtypeStruct((N, LAT), _f32),
               jax.ShapeDtypeStruct((N, LAT), _f32), _tab),
)
_stage4 = pl.pallas_call(
    _stage4_body,
    grid=_G,
    in_specs=[_bs_tab(), _bs_tab(), _vec_spec,
              _bs_full((D, HD)), _bs_full((D, HD)),
              _bs_full((1, HD)), _bs_full((1, HD)), _vec_spec],
    out_specs=_bs_tab(),
    out_shape=_tab)
_stage5 = pl.pallas_call(
    _stage5_body,
    grid=_G,
    in_specs=[_bs_tab(), _vec_spec],
    out_specs=_bs_rows(D),
    out_shape=jax.ShapeDtypeStruct((N, D), _f32))


def kernel(x, edge_index, W_h, b_h, W_mean, b_mean, W_ls, b_ls,
           W_dh, b_dh, W_do, b_do, eps):
    s = edge_index[0].astype(jnp.int32)
    r = edge_index[1].astype(jnp.int32)
    s2 = s.reshape(NC, NS, NCH, C)
    r2 = r.reshape(NC, NS, NCH, C)
    pad = ((0, 0), (0, PAD))
    s3 = jnp.pad(s.reshape(NS, EPS_), pad).reshape(NS, NCH3, C2)
    r3 = jnp.pad(r.reshape(NS, EPS_), pad,
                 constant_values=N).reshape(NS, NCH3, C2)
    degp = _deg_kernel(s2, r2).reshape(NC, 2, NPAD)
    dsa = degp[0, 0, :N, None]
    dsb = degp[1, 0, :N, None]
    dra = degp[0, 1, :N, None]
    drb = degp[1, 1, :N, None]
    bh = b_h[None, :]
    t1, rss, rrs, rsl, rrl = _stage1(
        x, W_h[:, :HD], W_h[:, HD:], bh[:, :HD], bh[:, HD:],
        dsa, dsb, dra, drb)
    A1 = _prop(t1, s3, r3)
    t23 = _stage2(A1, t1, rrs, rsl, W_mean, b_mean[None, :],
                  W_ls, b_ls[None, :])
    A23 = _prop(t23, s3, r3)
    bdh = b_dh[None, :]
    mean, log_std, t4 = _stage3(
        A23, rrl, eps, W_dh[:, :HD], W_dh[:, HD:], bdh[:, :HD], bdh[:, HD:],
        rss)
    A4 = _prop(t4, s3, r3)
    bdo = b_do[None, :]
    t5 = _stage4(A4, t4, rrs, W_do[:, :HD], W_do[:, HD:],
                 bdo[:, :HD], bdo[:, HD:], rsl)
    A5 = _prop(t5, s3, r3)
    out = _stage5(A5, rrl)
    return (mean, log_std, out)


# speedup vs baseline: 1.0978x; 1.0666x over previous
import jax
import jax.numpy as jnp
from jax import lax
from jax.experimental import pallas as pl
from jax.experimental.pallas import tpu as pltpu
from jax.experimental.pallas import tpu_sc as plsc

N = 10000
E = 320000
D = 128
LAT = 64
NC, NS, L = 2, 16, 16
NW = NC * NS
EPW = E // NW
C = 80
NCH = EPW // C
NBUF = 5
HD = D // NC
EPS_ = E // NS
NCH2 = EPS_ // C
C2 = 80
NCH3 = 250
PAD = NCH3 * C2 - EPS_
NGRP = NCH3 // NBUF
NPAIR = NGRP // 2
NTRASH = 8
ROWS_PW = N // NS
NPAD = 10240
DSL = NPAD // NS

_mesh = plsc.VectorSubcoreMesh(
    core_axis_name="c", subcore_axis_name="s", num_cores=NC, num_subcores=NS)


def _deg_body(s_hbm, r_hbm, out_hbm, s_idx, r_idx, zbuf, ones_buf,
              deg_s_sh, deg_r_sh, dsem):
    c = lax.axis_index("c")
    s = lax.axis_index("s")
    z16 = jnp.zeros((L,), jnp.float32)
    o16 = jnp.full((L,), 1.0, jnp.float32)
    for k in range(DSL // L):
        zbuf[pl.ds(k * L, L)] = z16
    for k in range(C // L):
        ones_buf[pl.ds(k * L, L)] = o16
    pltpu.sync_copy(zbuf, deg_s_sh.at[pl.ds(s * DSL, DSL)])
    pltpu.sync_copy(zbuf, deg_r_sh.at[pl.ds(s * DSL, DSL)])
    pltpu.sync_copy(s_hbm.at[c, s], s_idx)
    pltpu.sync_copy(r_hbm.at[c, s], r_idx)
    plsc.subcore_barrier()

    def body(g, carry):
        cps = []
        for b in range(5):
            j = g * 5 + b
            cps.append(pltpu.async_copy(
                ones_buf, deg_s_sh.at[s_idx.at[j]], dsem, add=True))
            cps.append(pltpu.async_copy(
                ones_buf, deg_r_sh.at[r_idx.at[j]], dsem, add=True))
        for cp in cps:
            cp.wait()
        return carry

    lax.fori_loop(0, NCH // 5, body, 0)
    plsc.subcore_barrier()
    pltpu.sync_copy(deg_s_sh.at[pl.ds(s * DSL, DSL)], out_hbm.at[c, 0, s])
    pltpu.sync_copy(deg_r_sh.at[pl.ds(s * DSL, DSL)], out_hbm.at[c, 1, s])


_deg_kernel = pl.kernel(
    _deg_body,
    out_type=jax.ShapeDtypeStruct((NC, 2, NS, DSL), jnp.float32),
    mesh=_mesh,
    scratch_types=[
        pltpu.VMEM((NCH, C), jnp.int32),
        pltpu.VMEM((NCH, C), jnp.int32),
        pltpu.VMEM((DSL,), jnp.float32),
        pltpu.VMEM((C,), jnp.float32),
        pltpu.VMEM_SHARED((NPAD,), jnp.float32),
        pltpu.VMEM_SHARED((NPAD,), jnp.float32),
        pltpu.SemaphoreType.DMA,
    ],
    compiler_params=pltpu.CompilerParams(use_tc_tiling_on_sc=False),
)


def _prop_body(tab_hbm, s_hbm, r_hbm, out_hbm,
               si0, ri0, si1, ri1,
               p00, p01, p02, p03, p04, p10, p11, p12, p13, p14,
               acc_sh, gsem0, gsem1, ssem0, ssem1, isem0, isem1):
    P = ((p00, p01, p02, p03, p04), (p10, p11, p12, p13, p14))
    SI = (si0, si1)
    RI = (ri0, ri1)
    gsem = (gsem0, gsem1)
    ssem = (ssem0, ssem1)
    isem = (isem0, isem1)
    c = lax.axis_index("c")
    s = lax.axis_index("s")
    z16 = jnp.zeros((L,), jnp.float32)

    def zs(i, carry):
        for k in range(HD // L):
            P[0][0][i, pl.ds(k * L, L)] = z16
        return carry

    lax.fori_loop(0, C2, zs, 0)
    base = s * ROWS_PW
    for k in range(ROWS_PW // C2):
        pltpu.sync_copy(P[0][0], acc_sh.at[pl.ds(base + k * C2, C2)])
    rem = ROWS_PW - (ROWS_PW // C2) * C2
    pltpu.sync_copy(P[0][0].at[pl.ds(0, rem)],
                    acc_sh.at[pl.ds(base + (ROWS_PW // C2) * C2, rem)])
    tab_c = tab_hbm.at[c]
    s_row = s_hbm.at[s]
    r_row = r_hbm.at[s]

    def fetch_idx(par, j0):
        pltpu.async_copy(s_row.at[pl.ds(j0, NBUF)], SI[par], isem[par])
        pltpu.async_copy(r_row.at[pl.ds(j0, NBUF)], RI[par], isem[par])

    def wait_idx(par):
        pltpu.make_async_copy(s_row.at[pl.ds(0, NBUF)], SI[par],
                              isem[par]).wait()
        pltpu.make_async_copy(r_row.at[pl.ds(0, NBUF)], RI[par],
                              isem[par]).wait()

    def fire_gathers(par):
        for b in range(NBUF):
            pltpu.async_copy(tab_c.at[SI[par].at[b]], P[par][b], gsem[par])

    def wait_gathers(par):
        for b in range(NBUF):
            pltpu.make_async_copy(tab_c.at[pl.ds(0, C2)], P[par][b],
                                  gsem[par]).wait()

    def do_scatters(par):
        for b in range(NBUF):
            pltpu.async_copy(
                P[par][b], acc_sh.at[RI[par].at[b]], ssem[par], add=True)

    def wait_scatters(par):
        for b in range(NBUF):
            pltpu.make_async_copy(tab_c.at[pl.ds(0, C2)], P[par][b],
                                  ssem[par]).wait()

    pltpu.sync_copy(s_row.at[pl.ds(0, NBUF)], SI[0])
    pltpu.sync_copy(r_row.at[pl.ds(0, NBUF)], RI[0])
    fetch_idx(1, NBUF)
    plsc.subcore_barrier()
    fire_gathers(0)
    JMAX = NCH3 - NBUF

    def pair(i, carry):
        a = 2 * i
        wait_gathers(0)
        wait_idx(1)
        fire_gathers(1)
        do_scatters(0)
        wait_scatters(0)
        fetch_idx(0, jnp.minimum((a + 2) * NBUF, JMAX))
        wait_gathers(1)
        wait_idx(0)
        fire_gathers(0)
        do_scatters(1)
        wait_scatters(1)
        fetch_idx(1, jnp.minimum((a + 3) * NBUF, JMAX))
        return carry

    lax.fori_loop(0, NPAIR, pair, 0)
    wait_gathers(0)
    wait_idx(1)
    plsc.subcore_barrier()
    pltpu.sync_copy(acc_sh.at[pl.ds(base, ROWS_PW)],
                    out_hbm.at[pl.ds(base, ROWS_PW), pl.ds(c * HD, HD)])


_prop_kernel = pl.kernel(
    _prop_body,
    out_type=jax.ShapeDtypeStruct((N, D), jnp.float32),
    mesh=_mesh,
    scratch_types=(
        [pltpu.VMEM((NBUF, C2), jnp.int32) for _ in range(4)]
        + [pltpu.VMEM((C2, HD), jnp.float32) for _ in range(2 * NBUF)]
        + [pltpu.VMEM_SHARED((N + NTRASH, HD), jnp.float32)]
        + [pltpu.SemaphoreType.DMA for _ in range(6)]
    ),
    compiler_params=pltpu.CompilerParams(use_tc_tiling_on_sc=False),
)


def _prop(tab, s2, r2):
    return _prop_kernel(tab, s2, r2)


def _elu(u):
    return jnp.where(u > 0, u, jnp.exp(u) - 1.0)


def _cat(A_ref):
    return jnp.concatenate([A_ref[0], A_ref[1]], axis=1)


def _mm(h, W_ref, b_ref):
    return jnp.dot(h, W_ref[...], preferred_element_type=jnp.float32) + b_ref[...]


def _stage1_body(x_ref, Wa_ref, Wb_ref, ba_ref, bb_ref, dsa, dsb, dra, drb,
                 t1_ref, rss_ref, rrs_ref, rsl_ref, rrl_ref):
    deg_s = dsa[...] + dsb[...]
    deg_r = dra[...] + drb[...]
    rss = lax.rsqrt(deg_s + 1.0)
    rss_ref[...] = rss
    rrs_ref[...] = lax.rsqrt(deg_r + 1.0)
    rsl_ref[...] = lax.rsqrt(jnp.maximum(deg_s, 1.0))
    rrl_ref[...] = lax.rsqrt(jnp.maximum(deg_r, 1.0))
    x = x_ref[...]
    t1_ref[0] = _elu(_mm(x, Wa_ref, ba_ref)) * rss
    t1_ref[1] = _elu(_mm(x, Wb_ref, bb_ref)) * rss


def _stage2_body(A_ref, t1_ref, rrs_ref, rsl_ref,
                 Wm_ref, bm_ref, Wl_ref, bl_ref, t23_ref):
    h = (A_ref[...] + _cat(t1_ref)) * rrs_ref[...]
    rsl = rsl_ref[...]
    t23_ref[0] = _mm(h, Wm_ref, bm_ref) * rsl
    t23_ref[1] = _mm(h, Wl_ref, bl_ref) * rsl


def _stage3_body(A_ref, rrl_ref, eps_ref, Wa_ref, Wb_ref, ba_ref, bb_ref,
                 rss_ref, mean_ref, ls_ref, t4_ref):
    rrl = rrl_ref[...]
    ml = A_ref[...] * rrl
    mean = ml[:, :LAT]
    lsd = ml[:, LAT:]
    mean_ref[...] = mean
    ls_ref[...] = lsd
    z = mean + jnp.exp(lsd) * eps_ref[...]
    rss = rss_ref[...]
    t4_ref[0] = _elu(_mm(z, Wa_ref, ba_ref)) * rss
    t4_ref[1] = _elu(_mm(z, Wb_ref, bb_ref)) * rss


def _stage4_body(A_ref, t4_ref, rrs_ref, Wa_ref, Wb_ref, ba_ref, bb_ref,
                 rsl_ref, t5_ref):
    d = (A_ref[...] + _cat(t4_ref)) * rrs_ref[...]
    rsl = rsl_ref[...]
    t5_ref[0] = _mm(d, Wa_ref, ba_ref) * rsl
    t5_ref[1] = _mm(d, Wb_ref, bb_ref) * rsl


def _stage5_body(A_ref, rrl_ref, out_ref):
    out_ref[...] = A_ref[...] * rrl_ref[...]


_f32 = jnp.float32
BN = 2000
_G = (N // BN,)
_vec = jax.ShapeDtypeStruct((N, 1), _f32)
_tab = jax.ShapeDtypeStruct((NC, N, HD), _f32)


def _bs_rows(w):
    return pl.BlockSpec((BN, w), lambda i: (i, 0))


def _bs_tab():
    return pl.BlockSpec((NC, BN, HD), lambda i: (0, i, 0))


def _bs_full(shape):
    return pl.BlockSpec(shape, lambda i: tuple(0 for _ in shape))


_vec_spec = _bs_rows(1)
_tab_out = jax.ShapeDtypeStruct((NC, N, HD), _f32)

_stage1 = pl.pallas_call(
    _stage1_body,
    grid=_G,
    in_specs=[_bs_rows(D), _bs_full((D, HD)), _bs_full((D, HD)),
              _bs_full((1, HD)), _bs_full((1, HD)),
              _vec_spec, _vec_spec, _vec_spec, _vec_spec],
    out_specs=(_bs_tab(), _vec_spec, _vec_spec, _vec_spec, _vec_spec),
    out_shape=(_tab, _vec, _vec, _vec, _vec))
_stage2 = pl.pallas_call(
    _stage2_body,
    grid=_G,
    in_specs=[_bs_rows(D), _bs_tab(), _vec_spec, _vec_spec,
              _bs_full((D, HD)), _bs_full((1, HD)),
              _bs_full((D, HD)), _bs_full((1, HD))],
    out_specs=_bs_tab(),
    out_shape=_tab)
_stage3 = pl.pallas_call(
    _stage3_body,
    grid=_G,
    in_specs=[_bs_rows(D), _vec_spec, _bs_rows(LAT),
              _bs_full((LAT, HD)), _bs_full((LAT, HD)),
              _bs_full((1, HD)), _bs_full((1, HD)), _vec_spec],
    out_specs=(_bs_rows(LAT), _bs_rows(LAT), _bs_tab()),
    out_shape=(jax.ShapeDtypeStruct((N, LAT), _f32),
               jax.ShapeDtypeStruct((N, LAT), _f32), _tab),
)
_stage4 = pl.pallas_call(
    _stage4_body,
    grid=_G,
    in_specs=[_bs_rows(D), _bs_tab(), _vec_spec,
              _bs_full((D, HD)), _bs_full((D, HD)),
              _bs_full((1, HD)), _bs_full((1, HD)), _vec_spec],
    out_specs=_bs_tab(),
    out_shape=_tab)
_stage5 = pl.pallas_call(
    _stage5_body,
    grid=_G,
    in_specs=[_bs_rows(D), _vec_spec],
    out_specs=_bs_rows(D),
    out_shape=jax.ShapeDtypeStruct((N, D), _f32))


def kernel(x, edge_index, W_h, b_h, W_mean, b_mean, W_ls, b_ls,
           W_dh, b_dh, W_do, b_do, eps):
    s = edge_index[0].astype(jnp.int32)
    r = edge_index[1].astype(jnp.int32)
    s2 = s.reshape(NC, NS, NCH, C)
    r2 = r.reshape(NC, NS, NCH, C)
    pad = ((0, 0), (0, PAD))
    s3 = jnp.pad(s.reshape(NS, EPS_), pad).reshape(NS, NCH3, C2)
    r3 = jnp.pad(r.reshape(NS, EPS_), pad,
                 constant_values=N).reshape(NS, NCH3, C2)
    degp = _deg_kernel(s2, r2).reshape(NC, 2, NPAD)
    dsa = degp[0, 0, :N, None]
    dsb = degp[1, 0, :N, None]
    dra = degp[0, 1, :N, None]
    drb = degp[1, 1, :N, None]
    bh = b_h[None, :]
    t1, rss, rrs, rsl, rrl = _stage1(
        x, W_h[:, :HD], W_h[:, HD:], bh[:, :HD], bh[:, HD:],
        dsa, dsb, dra, drb)
    A1 = _prop(t1, s3, r3)
    t23 = _stage2(A1, t1, rrs, rsl, W_mean, b_mean[None, :],
                  W_ls, b_ls[None, :])
    A23 = _prop(t23, s3, r3)
    bdh = b_dh[None, :]
    mean, log_std, t4 = _stage3(
        A23, rrl, eps, W_dh[:, :HD], W_dh[:, HD:], bdh[:, :HD], bdh[:, HD:],
        rss)
    A4 = _prop(t4, s3, r3)
    bdo = b_do[None, :]
    t5 = _stage4(A4, t4, rrs, W_do[:, :HD], W_do[:, HD:],
                 bdo[:, :HD], bdo[:, HD:], rsl)
    A5 = _prop(t5, s3, r3)
    out = _stage5(A5, rrl)
    return (mean, log_std, out)


# speedup vs baseline: 1.1786x; 1.0736x over previous
import jax
import jax.numpy as jnp
from jax import lax
from jax.experimental import pallas as pl
from jax.experimental.pallas import tpu as pltpu
from jax.experimental.pallas import tpu_sc as plsc

N = 10000
E = 320000
D = 128
LAT = 64
NC, NS, L = 2, 16, 16
NW = NC * NS
EPW = E // NW
C = 80
NCH = EPW // C
NBUF = 5
HD = D // NC
EPS_ = E // NS
NCH2 = EPS_ // C
C2 = 80
NCH3 = 250
PAD = NCH3 * C2 - EPS_
NGRP = NCH3 // NBUF
NPAIR = NGRP // 2
NTRASH = 8
ROWS_PW = N // NS
NPAD = 10240
DSL = NPAD // NS

_mesh = plsc.VectorSubcoreMesh(
    core_axis_name="c", subcore_axis_name="s", num_cores=NC, num_subcores=NS)


def _deg_body(s_hbm, r_hbm, out_hbm, s_idx, r_idx, zbuf, ones_buf,
              deg_s_sh, deg_r_sh, dsem):
    c = lax.axis_index("c")
    s = lax.axis_index("s")
    z16 = jnp.zeros((L,), jnp.float32)
    o16 = jnp.full((L,), 1.0, jnp.float32)
    for k in range(DSL // L):
        zbuf[pl.ds(k * L, L)] = z16
    for k in range(C // L):
        ones_buf[pl.ds(k * L, L)] = o16
    pltpu.sync_copy(zbuf, deg_s_sh.at[pl.ds(s * DSL, DSL)])
    pltpu.sync_copy(zbuf, deg_r_sh.at[pl.ds(s * DSL, DSL)])
    pltpu.sync_copy(s_hbm.at[c, s], s_idx)
    pltpu.sync_copy(r_hbm.at[c, s], r_idx)
    plsc.subcore_barrier()

    def body(g, carry):
        cps = []
        for b in range(5):
            j = g * 5 + b
            cps.append(pltpu.async_copy(
                ones_buf, deg_s_sh.at[s_idx.at[j]], dsem, add=True))
            cps.append(pltpu.async_copy(
                ones_buf, deg_r_sh.at[r_idx.at[j]], dsem, add=True))
        for cp in cps:
            cp.wait()
        return carry

    lax.fori_loop(0, NCH // 5, body, 0)
    plsc.subcore_barrier()
    pltpu.sync_copy(deg_s_sh.at[pl.ds(s * DSL, DSL)], out_hbm.at[c, 0, s])
    pltpu.sync_copy(deg_r_sh.at[pl.ds(s * DSL, DSL)], out_hbm.at[c, 1, s])


_deg_kernel = pl.kernel(
    _deg_body,
    out_type=jax.ShapeDtypeStruct((NC, 2, NS, DSL), jnp.float32),
    mesh=_mesh,
    scratch_types=[
        pltpu.VMEM((NCH, C), jnp.int32),
        pltpu.VMEM((NCH, C), jnp.int32),
        pltpu.VMEM((DSL,), jnp.float32),
        pltpu.VMEM((C,), jnp.float32),
        pltpu.VMEM_SHARED((NPAD,), jnp.float32),
        pltpu.VMEM_SHARED((NPAD,), jnp.float32),
        pltpu.SemaphoreType.DMA,
    ],
    compiler_params=pltpu.CompilerParams(use_tc_tiling_on_sc=False),
)


def _prop_body(tab_hbm, s_hbm, r_hbm, out_hbm,
               si0, ri0, si1, ri1, sj0, sj1,
               p00, p01, p02, p03, p04, p10, p11, p12, p13, p14,
               acc_sh, gsem0, gsem1, ssem0, ssem1, isem0, isem1):
    P = ((p00, p01, p02, p03, p04), (p10, p11, p12, p13, p14))
    SI = (si0, si1)
    RI = (ri0, ri1)
    SJ = (sj0, sj1)
    gsem = (gsem0, gsem1)
    ssem = (ssem0, ssem1)
    isem = (isem0, isem1)
    c = lax.axis_index("c")
    s = lax.axis_index("s")
    z16 = jnp.zeros((L,), jnp.float32)

    def zs(i, carry):
        for k in range(HD // L):
            P[0][0][i, pl.ds(k * L, L)] = z16
        return carry

    lax.fori_loop(0, C2, zs, 0)
    base = s * ROWS_PW
    for k in range(ROWS_PW // C2):
        pltpu.sync_copy(P[0][0], acc_sh.at[pl.ds(base + k * C2, C2)])
    rem = ROWS_PW - (ROWS_PW // C2) * C2
    pltpu.sync_copy(P[0][0].at[pl.ds(0, rem)],
                    acc_sh.at[pl.ds(base + (ROWS_PW // C2) * C2, rem)])
    s_row = s_hbm.at[s]
    r_row = r_hbm.at[s]

    def fetch_idx(par, j0):
        pltpu.async_copy(s_row.at[pl.ds(j0, NBUF)], SI[par], isem[par])
        pltpu.async_copy(r_row.at[pl.ds(j0, NBUF)], RI[par], isem[par])

    def wait_idx(par):
        pltpu.make_async_copy(s_row.at[pl.ds(0, NBUF)], SI[par],
                              isem[par]).wait()
        pltpu.make_async_copy(r_row.at[pl.ds(0, NBUF)], RI[par],
                              isem[par]).wait()
        for b in range(NBUF):
            for k in range(C2 // L):
                v = SI[par][b, pl.ds(k * L, L)]
                SJ[par][b, pl.ds(k * L, L)] = v * 2 + c

    def fire_gathers(par):
        for b in range(NBUF):
            pltpu.async_copy(tab_hbm.at[SJ[par].at[b]], P[par][b], gsem[par])

    def wait_gathers(par):
        for b in range(NBUF):
            pltpu.make_async_copy(tab_hbm.at[pl.ds(0, C2)], P[par][b],
                                  gsem[par]).wait()

    def do_scatters(par):
        for b in range(NBUF):
            pltpu.async_copy(
                P[par][b], acc_sh.at[RI[par].at[b]], ssem[par], add=True)

    def wait_scatters(par):
        for b in range(NBUF):
            pltpu.make_async_copy(tab_hbm.at[pl.ds(0, C2)], P[par][b],
                                  ssem[par]).wait()

    pltpu.sync_copy(s_row.at[pl.ds(0, NBUF)], SI[0])
    pltpu.sync_copy(r_row.at[pl.ds(0, NBUF)], RI[0])
    for b in range(NBUF):
        for k in range(C2 // L):
            v = SI[0][b, pl.ds(k * L, L)]
            SJ[0][b, pl.ds(k * L, L)] = v * 2 + c
    fetch_idx(1, NBUF)
    plsc.subcore_barrier()
    fire_gathers(0)
    JMAX = NCH3 - NBUF

    def pair(i, carry):
        a = 2 * i
        wait_gathers(0)
        wait_idx(1)
        fire_gathers(1)
        do_scatters(0)
        wait_scatters(0)
        fetch_idx(0, jnp.minimum((a + 2) * NBUF, JMAX))
        wait_gathers(1)
        wait_idx(0)
        fire_gathers(0)
        do_scatters(1)
        wait_scatters(1)
        fetch_idx(1, jnp.minimum((a + 3) * NBUF, JMAX))
        return carry

    lax.fori_loop(0, NPAIR, pair, 0)
    wait_gathers(0)
    wait_idx(1)
    plsc.subcore_barrier()
    pltpu.sync_copy(acc_sh.at[pl.ds(base, ROWS_PW)],
                    out_hbm.at[pl.ds(base, ROWS_PW), pl.ds(c * HD, HD)])


_prop_kernel = pl.kernel(
    _prop_body,
    out_type=jax.ShapeDtypeStruct((N, D), jnp.float32),
    mesh=_mesh,
    scratch_types=(
        [pltpu.VMEM((NBUF, C2), jnp.int32) for _ in range(6)]
        + [pltpu.VMEM((C2, HD), jnp.float32) for _ in range(2 * NBUF)]
        + [pltpu.VMEM_SHARED((N + NTRASH, HD), jnp.float32)]
        + [pltpu.SemaphoreType.DMA for _ in range(6)]
    ),
    compiler_params=pltpu.CompilerParams(use_tc_tiling_on_sc=False),
)


def _prop(tab, s2, r2):
    return _prop_kernel(tab.reshape(NC * N, HD), s2, r2)


def _elu(u):
    return jnp.where(u > 0, u, jnp.exp(u) - 1.0)


def _mm(h, W_ref, b_ref):
    return jnp.dot(h, W_ref[...], preferred_element_type=jnp.float32) + b_ref[...]


def _stage1_body(x_ref, W_ref, b_ref, dsa, dsb, dra, drb,
                 t1_ref, rss_ref, rrs_ref, rsl_ref, rrl_ref):
    deg_s = dsa[...] + dsb[...]
    deg_r = dra[...] + drb[...]
    rss = lax.rsqrt(deg_s + 1.0)
    rss_ref[...] = rss
    rrs_ref[...] = lax.rsqrt(deg_r + 1.0)
    rsl_ref[...] = lax.rsqrt(jnp.maximum(deg_s, 1.0))
    rrl_ref[...] = lax.rsqrt(jnp.maximum(deg_r, 1.0))
    t1_ref[...] = _elu(_mm(x_ref[...], W_ref, b_ref)) * rss


def _stage2_body(A_ref, t1_ref, rrs_ref, rsl_ref, W2_ref, b2_ref, t23_ref):
    h = (A_ref[...] + t1_ref[...]) * rrs_ref[...]
    t23_ref[...] = _mm(h, W2_ref, b2_ref) * rsl_ref[...]


def _stage3_body(A_ref, rrl_ref, eps_ref, Wd_ref, bd_ref,
                 rss_ref, mean_ref, ls_ref, t4_ref):
    ml = A_ref[...] * rrl_ref[...]
    mean = ml[:, :LAT]
    lsd = ml[:, LAT:]
    mean_ref[...] = mean
    ls_ref[...] = lsd
    z = mean + jnp.exp(lsd) * eps_ref[...]
    t4_ref[...] = _elu(_mm(z, Wd_ref, bd_ref)) * rss_ref[...]


def _stage4_body(A_ref, t4_ref, rrs_ref, Wo_ref, bo_ref, rsl_ref, t5_ref):
    d = (A_ref[...] + t4_ref[...]) * rrs_ref[...]
    t5_ref[...] = _mm(d, Wo_ref, bo_ref) * rsl_ref[...]


def _stage5_body(A_ref, rrl_ref, out_ref):
    out_ref[...] = A_ref[...] * rrl_ref[...]


_f32 = jnp.float32
BN = 2000
_G = (N // BN,)
_vec = jax.ShapeDtypeStruct((N, 1), _f32)
_tab = jax.ShapeDtypeStruct((N, D), _f32)


def _bs_rows(w):
    return pl.BlockSpec((BN, w), lambda i: (i, 0))


def _bs_full(shape):
    return pl.BlockSpec(shape, lambda i: tuple(0 for _ in shape))


_vec_spec = _bs_rows(1)

_stage1 = pl.pallas_call(
    _stage1_body,
    grid=_G,
    in_specs=[_bs_rows(D), _bs_full((D, D)), _bs_full((1, D)),
              _vec_spec, _vec_spec, _vec_spec, _vec_spec],
    out_specs=(_bs_rows(D), _vec_spec, _vec_spec, _vec_spec, _vec_spec),
    out_shape=(_tab, _vec, _vec, _vec, _vec))
_stage2 = pl.pallas_call(
    _stage2_body,
    grid=_G,
    in_specs=[_bs_rows(D), _bs_rows(D), _vec_spec, _vec_spec,
              _bs_full((D, D)), _bs_full((1, D))],
    out_specs=_bs_rows(D),
    out_shape=_tab)
_stage3 = pl.pallas_call(
    _stage3_body,
    grid=_G,
    in_specs=[_bs_rows(D), _vec_spec, _bs_rows(LAT),
              _bs_full((LAT, D)), _bs_full((1, D)), _vec_spec],
    out_specs=(_bs_rows(LAT), _bs_rows(LAT), _bs_rows(D)),
    out_shape=(jax.ShapeDtypeStruct((N, LAT), _f32),
               jax.ShapeDtypeStruct((N, LAT), _f32), _tab),
)
_stage4 = pl.pallas_call(
    _stage4_body,
    grid=_G,
    in_specs=[_bs_rows(D), _bs_rows(D), _vec_spec,
              _bs_full((D, D)), _bs_full((1, D)), _vec_spec],
    out_specs=_bs_rows(D),
    out_shape=_tab)
_stage5 = pl.pallas_call(
    _stage5_body,
    grid=_G,
    in_specs=[_bs_rows(D), _vec_spec],
    out_specs=_bs_rows(D),
    out_shape=_tab)


def kernel(x, edge_index, W_h, b_h, W_mean, b_mean, W_ls, b_ls,
           W_dh, b_dh, W_do, b_do, eps):
    s = edge_index[0].astype(jnp.int32)
    r = edge_index[1].astype(jnp.int32)
    s2 = s.reshape(NC, NS, NCH, C)
    r2 = r.reshape(NC, NS, NCH, C)
    pad = ((0, 0), (0, PAD))
    s3 = jnp.pad(s.reshape(NS, EPS_), pad).reshape(NS, NCH3, C2)
    r3 = jnp.pad(r.reshape(NS, EPS_), pad,
                 constant_values=N).reshape(NS, NCH3, C2)
    degp = _deg_kernel(s2, r2).reshape(NC, 2, NPAD)
    dsa = degp[0, 0, :N, None]
    dsb = degp[1, 0, :N, None]
    dra = degp[0, 1, :N, None]
    drb = degp[1, 1, :N, None]
    W2 = jnp.concatenate([W_mean, W_ls], axis=1)
    b2 = jnp.concatenate([b_mean, b_ls])[None, :]
    t1, rss, rrs, rsl, rrl = _stage1(
        x, W_h, b_h[None, :], dsa, dsb, dra, drb)
    A1 = _prop(t1, s3, r3)
    t23 = _stage2(A1, t1, rrs, rsl, W2, b2)
    A23 = _prop(t23, s3, r3)
    mean, log_std, t4 = _stage3(A23, rrl, eps, W_dh, b_dh[None, :], rss)
    A4 = _prop(t4, s3, r3)
    t5 = _stage4(A4, t4, rrs, W_do, b_do[None, :], rsl)
    A5 = _prop(t5, s3, r3)
    out = _stage5(A5, rrl)
    return (mean, log_std, out)


# speedup vs baseline: 1.1799x; 1.0011x over previous
import jax
import jax.numpy as jnp
from jax import lax
from jax.experimental import pallas as pl
from jax.experimental.pallas import tpu as pltpu
from jax.experimental.pallas import tpu_sc as plsc

N = 10000
E = 320000
D = 128
LAT = 64
NC, NS, L = 2, 16, 16
NW = NC * NS
EPW = E // NW
C = 80
NCH = EPW // C
NBUF = 5
HD = D // NC
EPS_ = E // NS
C2 = 80
NCH3 = EPS_ // C2
PAD = NCH3 * C2 - EPS_
NGRP = NCH3 // NBUF
NPAIR = NGRP // 2
NTRASH = 8
ROWS_PW = N // NS
NPAD = 10240
DSL = NPAD // NS

_mesh = plsc.VectorSubcoreMesh(
    core_axis_name="c", subcore_axis_name="s", num_cores=NC, num_subcores=NS)


def _deg_body(s_hbm, r_hbm, out_hbm, s_idx, r_idx, zbuf, ones_buf,
              deg_s_sh, deg_r_sh, dsem):
    c = lax.axis_index("c")
    s = lax.axis_index("s")
    z16 = jnp.zeros((L,), jnp.float32)
    o16 = jnp.full((L,), 1.0, jnp.float32)
    for k in range(DSL // L):
        zbuf[pl.ds(k * L, L)] = z16
    for k in range(C // L):
        ones_buf[pl.ds(k * L, L)] = o16
    pltpu.sync_copy(zbuf, deg_s_sh.at[pl.ds(s * DSL, DSL)])
    pltpu.sync_copy(zbuf, deg_r_sh.at[pl.ds(s * DSL, DSL)])
    pltpu.sync_copy(s_hbm.at[c, s], s_idx)
    pltpu.sync_copy(r_hbm.at[c, s], r_idx)
    plsc.subcore_barrier()

    def body(g, carry):
        cps = []
        for b in range(5):
            j = g * 5 + b
            cps.append(pltpu.async_copy(
                ones_buf, deg_s_sh.at[s_idx.at[j]], dsem, add=True))
            cps.append(pltpu.async_copy(
                ones_buf, deg_r_sh.at[r_idx.at[j]], dsem, add=True))
        for cp in cps:
            cp.wait()
        return carry

    lax.fori_loop(0, NCH // 5, body, 0)
    plsc.subcore_barrier()
    pltpu.sync_copy(deg_s_sh.at[pl.ds(s * DSL, DSL)], out_hbm.at[c, 0, s])
    pltpu.sync_copy(deg_r_sh.at[pl.ds(s * DSL, DSL)], out_hbm.at[c, 1, s])


_deg_kernel = pl.kernel(
    _deg_body,
    out_type=jax.ShapeDtypeStruct((NC, 2, NS, DSL), jnp.float32),
    mesh=_mesh,
    scratch_types=[
        pltpu.VMEM((NCH, C), jnp.int32),
        pltpu.VMEM((NCH, C), jnp.int32),
        pltpu.VMEM((DSL,), jnp.float32),
        pltpu.VMEM((C,), jnp.float32),
        pltpu.VMEM_SHARED((NPAD,), jnp.float32),
        pltpu.VMEM_SHARED((NPAD,), jnp.float32),
        pltpu.SemaphoreType.DMA,
    ],
    compiler_params=pltpu.CompilerParams(use_tc_tiling_on_sc=False),
)


def _prop_body(tab_hbm, s_hbm, r_hbm, out_hbm,
               si0, ri0, si1, ri1, sj0, sj1,
               p00, p01, p02, p03, p04, p10, p11, p12, p13, p14,
               acc_sh, gsem0, gsem1, ssem0, ssem1, isem0, isem1):
    P = ((p00, p01, p02, p03, p04), (p10, p11, p12, p13, p14))
    SI = (si0, si1)
    RI = (ri0, ri1)
    SJ = (sj0, sj1)
    gsem = (gsem0, gsem1)
    ssem = (ssem0, ssem1)
    isem = (isem0, isem1)
    c = lax.axis_index("c")
    s = lax.axis_index("s")
    z16 = jnp.zeros((L,), jnp.float32)

    def zs(i, carry):
        for k in range(HD // L):
            P[0][0][i, pl.ds(k * L, L)] = z16
        return carry

    lax.fori_loop(0, C2, zs, 0)
    base = s * ROWS_PW
    for k in range(ROWS_PW // C2):
        pltpu.sync_copy(P[0][0], acc_sh.at[pl.ds(base + k * C2, C2)])
    rem = ROWS_PW - (ROWS_PW // C2) * C2
    pltpu.sync_copy(P[0][0].at[pl.ds(0, rem)],
                    acc_sh.at[pl.ds(base + (ROWS_PW // C2) * C2, rem)])
    s_row = s_hbm.at[s]
    r_row = r_hbm.at[s]

    def fetch_idx(par, j0):
        pltpu.async_copy(s_row.at[pl.ds(j0, NBUF)], SI[par], isem[par])
        pltpu.async_copy(r_row.at[pl.ds(j0, NBUF)], RI[par], isem[par])

    def wait_idx(par):
        pltpu.make_async_copy(s_row.at[pl.ds(0, NBUF)], SI[par],
                              isem[par]).wait()
        pltpu.make_async_copy(r_row.at[pl.ds(0, NBUF)], RI[par],
                              isem[par]).wait()
        for b in range(NBUF):
            for k in range(C2 // L):
                v = SI[par][b, pl.ds(k * L, L)]
                SJ[par][b, pl.ds(k * L, L)] = v * 2 + c

    def fire_gathers(par):
        for b in range(NBUF):
            pltpu.async_copy(tab_hbm.at[SJ[par].at[b]], P[par][b], gsem[par])

    def wait_gathers(par):
        for b in range(NBUF):
            pltpu.make_async_copy(tab_hbm.at[pl.ds(0, C2)], P[par][b],
                                  gsem[par]).wait()

    def do_scatters(par):
        for b in range(NBUF):
            pltpu.async_copy(
                P[par][b], acc_sh.at[RI[par].at[b]], ssem[par], add=True)

    def wait_scatters(par):
        for b in range(NBUF):
            pltpu.make_async_copy(tab_hbm.at[pl.ds(0, C2)], P[par][b],
                                  ssem[par]).wait()

    pltpu.sync_copy(s_row.at[pl.ds(0, NBUF)], SI[0])
    pltpu.sync_copy(r_row.at[pl.ds(0, NBUF)], RI[0])
    for b in range(NBUF):
        for k in range(C2 // L):
            v = SI[0][b, pl.ds(k * L, L)]
            SJ[0][b, pl.ds(k * L, L)] = v * 2 + c
    fetch_idx(1, NBUF)
    plsc.subcore_barrier()
    fire_gathers(0)
    JMAX = NCH3 - NBUF

    def pair(i, carry):
        a = 2 * i
        wait_gathers(0)
        wait_idx(1)
        fire_gathers(1)
        do_scatters(0)
        wait_scatters(0)
        fetch_idx(0, jnp.minimum((a + 2) * NBUF, JMAX))
        wait_gathers(1)
        wait_idx(0)
        fire_gathers(0)
        do_scatters(1)
        wait_scatters(1)
        fetch_idx(1, jnp.minimum((a + 3) * NBUF, JMAX))
        return carry

    lax.fori_loop(0, NPAIR, pair, 0)
    wait_gathers(0)
    wait_idx(1)
    plsc.subcore_barrier()
    pltpu.sync_copy(acc_sh.at[pl.ds(base, ROWS_PW)],
                    out_hbm.at[pl.ds(base, ROWS_PW), pl.ds(c * HD, HD)])


_prop_kernel = pl.kernel(
    _prop_body,
    out_type=jax.ShapeDtypeStruct((N, D), jnp.float32),
    mesh=_mesh,
    scratch_types=(
        [pltpu.VMEM((NBUF, C2), jnp.int32) for _ in range(6)]
        + [pltpu.VMEM((C2, HD), jnp.float32) for _ in range(2 * NBUF)]
        + [pltpu.VMEM_SHARED((N + NTRASH, HD), jnp.float32)]
        + [pltpu.SemaphoreType.DMA for _ in range(6)]
    ),
    compiler_params=pltpu.CompilerParams(use_tc_tiling_on_sc=False),
)


def _prop(tab, s2, r2):
    return _prop_kernel(tab.reshape(NC * N, HD), s2, r2)


def _elu(u):
    return jnp.where(u > 0, u, jnp.exp(u) - 1.0)


def _mm(h, W_ref, b_ref):
    return jnp.dot(h, W_ref[...], preferred_element_type=jnp.float32) + b_ref[...]


def _stage1_body(x_ref, W_ref, b_ref, dsa, dsb, dra, drb,
                 t1_ref, rss_ref, rrs_ref, rsl_ref, rrl_ref):
    deg_s = dsa[...] + dsb[...]
    deg_r = dra[...] + drb[...]
    rss = lax.rsqrt(deg_s + 1.0)
    rss_ref[...] = rss
    rrs_ref[...] = lax.rsqrt(deg_r + 1.0)
    rsl_ref[...] = lax.rsqrt(jnp.maximum(deg_s, 1.0))
    rrl_ref[...] = lax.rsqrt(jnp.maximum(deg_r, 1.0))
    t1_ref[...] = _elu(_mm(x_ref[...], W_ref, b_ref)) * rss


def _stage2_body(A_ref, t1_ref, rrs_ref, rsl_ref, W2_ref, b2_ref, t23_ref):
    h = (A_ref[...] + t1_ref[...]) * rrs_ref[...]
    t23_ref[...] = _mm(h, W2_ref, b2_ref) * rsl_ref[...]


def _stage3_body(A_ref, rrl_ref, eps_ref, Wd_ref, bd_ref,
                 rss_ref, mean_ref, ls_ref, t4_ref):
    ml = A_ref[...] * rrl_ref[...]
    mean = ml[:, :LAT]
    lsd = ml[:, LAT:]
    mean_ref[...] = mean
    ls_ref[...] = lsd
    z = mean + jnp.exp(lsd) * eps_ref[...]
    t4_ref[...] = _elu(_mm(z, Wd_ref, bd_ref)) * rss_ref[...]


def _stage4_body(A_ref, t4_ref, rrs_ref, Wo_ref, bo_ref, rsl_ref, t5_ref):
    d = (A_ref[...] + t4_ref[...]) * rrs_ref[...]
    t5_ref[...] = _mm(d, Wo_ref, bo_ref) * rsl_ref[...]


def _stage5_body(A_ref, rrl_ref, out_ref):
    out_ref[...] = A_ref[...] * rrl_ref[...]


_f32 = jnp.float32
BN = 2000
_G = (N // BN,)
_vec = jax.ShapeDtypeStruct((N, 1), _f32)
_tab = jax.ShapeDtypeStruct((N, D), _f32)


def _bs_rows(w):
    return pl.BlockSpec((BN, w), lambda i: (i, 0))


def _bs_full(shape):
    return pl.BlockSpec(shape, lambda i: tuple(0 for _ in shape))


_vec_spec = _bs_rows(1)

_stage1 = pl.pallas_call(
    _stage1_body,
    grid=_G,
    in_specs=[_bs_rows(D), _bs_full((D, D)), _bs_full((1, D)),
              _vec_spec, _vec_spec, _vec_spec, _vec_spec],
    out_specs=(_bs_rows(D), _vec_spec, _vec_spec, _vec_spec, _vec_spec),
    out_shape=(_tab, _vec, _vec, _vec, _vec))
_stage2 = pl.pallas_call(
    _stage2_body,
    grid=_G,
    in_specs=[_bs_rows(D), _bs_rows(D), _vec_spec, _vec_spec,
              _bs_full((D, D)), _bs_full((1, D))],
    out_specs=_bs_rows(D),
    out_shape=_tab)
_stage3 = pl.pallas_call(
    _stage3_body,
    grid=_G,
    in_specs=[_bs_rows(D), _vec_spec, _bs_rows(LAT),
              _bs_full((LAT, D)), _bs_full((1, D)), _vec_spec],
    out_specs=(_bs_rows(LAT), _bs_rows(LAT), _bs_rows(D)),
    out_shape=(jax.ShapeDtypeStruct((N, LAT), _f32),
               jax.ShapeDtypeStruct((N, LAT), _f32), _tab),
)
_stage4 = pl.pallas_call(
    _stage4_body,
    grid=_G,
    in_specs=[_bs_rows(D), _bs_rows(D), _vec_spec,
              _bs_full((D, D)), _bs_full((1, D)), _vec_spec],
    out_specs=_bs_rows(D),
    out_shape=_tab)
_stage5 = pl.pallas_call(
    _stage5_body,
    grid=_G,
    in_specs=[_bs_rows(D), _vec_spec],
    out_specs=_bs_rows(D),
    out_shape=_tab)


def kernel(x, edge_index, W_h, b_h, W_mean, b_mean, W_ls, b_ls,
           W_dh, b_dh, W_do, b_do, eps):
    s = edge_index[0].astype(jnp.int32)
    r = edge_index[1].astype(jnp.int32)
    s2 = s.reshape(NC, NS, NCH, C)
    r2 = r.reshape(NC, NS, NCH, C)
    pad = ((0, 0), (0, PAD))
    s3 = jnp.pad(s.reshape(NS, EPS_), pad).reshape(NS, NCH3, C2)
    r3 = jnp.pad(r.reshape(NS, EPS_), pad,
                 constant_values=N).reshape(NS, NCH3, C2)
    degp = _deg_kernel(s2, r2).reshape(NC, 2, NPAD)
    dsa = degp[0, 0, :N, None]
    dsb = degp[1, 0, :N, None]
    dra = degp[0, 1, :N, None]
    drb = degp[1, 1, :N, None]
    W2 = jnp.concatenate([W_mean, W_ls], axis=1)
    b2 = jnp.concatenate([b_mean, b_ls])[None, :]
    t1, rss, rrs, rsl, rrl = _stage1(
        x, W_h, b_h[None, :], dsa, dsb, dra, drb)
    A1 = _prop(t1, s3, r3)
    t23 = _stage2(A1, t1, rrs, rsl, W2, b2)
    A23 = _prop(t23, s3, r3)
    mean, log_std, t4 = _stage3(A23, rrl, eps, W_dh, b_dh[None, :], rss)
    A4 = _prop(t4, s3, r3)
    t5 = _stage4(A4, t4, rrs, W_do, b_do[None, :], rsl)
    A5 = _prop(t5, s3, r3)
    out = _stage5(A5, rrl)
    return (mean, log_std, out)
